# speedup vs baseline: 1.0142x; 1.0142x over previous
import functools

import jax
import jax.numpy as jnp
from jax import lax
from jax.experimental import pallas as pl
from jax.experimental.pallas import tpu as pltpu
from jax.experimental.pallas import tpu_sc as plsc

NUM_CORES = 2
NUM_SUBCORES = 16
NW = NUM_CORES * NUM_SUBCORES
LANES = 16

CHUNK = 128
CHUNKS_PER_WORKER = 25
NP = NW * CHUNKS_PER_WORKER * CHUNK
HIDDEN = 128
VREGS_PER_ROW = HIDDEN // LANES


def _sc_body(xT_hbm, w0_hbm, w1_hbm, w2_hbm, out_hbm,
             idx_v, buf0, buf1, buf2, sem):
    wid = lax.axis_index("s") * NUM_CORES + lax.axis_index("c")

    def chunk_step(i, carry):
        base = (wid * CHUNKS_PER_WORKER + i) * CHUNK
        pltpu.sync_copy(xT_hbm.at[:, pl.ds(base, CHUNK)], idx_v)
        c0 = pltpu.async_copy(w0_hbm.at[idx_v.at[0]], buf0, sem)
        c1 = pltpu.async_copy(w1_hbm.at[idx_v.at[1]], buf1, sem)
        c2 = pltpu.async_copy(w2_hbm.at[idx_v.at[2]], buf2, sem)
        c0.wait()
        c1.wait()
        c2.wait()

        def row_step(r, carry2):
            for j in range(VREGS_PER_ROW):
                sl = pl.ds(j * LANES, LANES)
                buf0[r, sl] = buf0[r, sl] + buf1[r, sl] + buf2[r, sl]
            return carry2

        lax.fori_loop(0, CHUNK, row_step, 0, unroll=2)

        pltpu.sync_copy(buf0, out_hbm.at[pl.ds(base, CHUNK), :])
        return carry

    lax.fori_loop(0, CHUNKS_PER_WORKER, chunk_step, 0)


@jax.jit
def _encode(xT, w0, w1, w2):
    mesh = plsc.VectorSubcoreMesh(core_axis_name="c", subcore_axis_name="s")
    return pl.kernel(
        _sc_body,
        out_type=jax.ShapeDtypeStruct((NP, HIDDEN), jnp.float32),
        mesh=mesh,
        scratch_types=[
            pltpu.VMEM((3, CHUNK), jnp.int32),
            pltpu.VMEM((CHUNK, HIDDEN), jnp.float32),
            pltpu.VMEM((CHUNK, HIDDEN), jnp.float32),
            pltpu.VMEM((CHUNK, HIDDEN), jnp.float32),
            pltpu.SemaphoreType.DMA,
        ],
    )(xT, w0, w1, w2)


def kernel(x, W0, W1, W2):
    if x.ndim == 1:
        x = x[:, None]
    n = x.shape[0]
    xT = jnp.zeros((3, NP), jnp.int32).at[:, :n].set(x.T.astype(jnp.int32))
    out = _encode(xT, W0, W1, W2)
    return out[:n]


# speedup vs baseline: 2.8389x; 2.7992x over previous
import functools

import jax
import jax.numpy as jnp
from jax import lax
from jax.experimental import pallas as pl
from jax.experimental.pallas import tpu as pltpu
from jax.experimental.pallas import tpu_sc as plsc

NUM_CORES = 2
NUM_SUBCORES = 16
NW = NUM_CORES * NUM_SUBCORES

T = 26
HIDDEN = 128
CHUNK = 128
CHUNKS_PER_WORKER = 25
ROWS_PER_WORKER = CHUNK * CHUNKS_PER_WORKER
NP = NW * ROWS_PER_WORKER


def _combine_body(w0_ref, w1_ref, w2_ref, xt_ref, s_ref, cidx_ref):
    w0, w1, w2 = w0_ref[...], w1_ref[...], w2_ref[...]
    s01 = (w0[:, None, :] + w1[None, :, :]).reshape(T * T, HIDDEN)
    s_ref[...] = (s01[:, None, :] + w2[None, :, :]).reshape(T * T * T, HIDDEN)
    xt = xt_ref[...]
    cidx_ref[...] = (T * T) * xt[0] + T * xt[1] + xt[2]


@functools.partial(jax.jit, static_argnames=())
def _combine(w0, w1, w2, xt3):
    return pl.pallas_call(
        _combine_body,
        out_shape=(
            jax.ShapeDtypeStruct((T * T * T, HIDDEN), jnp.float32),
            jax.ShapeDtypeStruct(xt3.shape[1:], jnp.int32),
        ),
    )(w0, w1, w2, xt3)


def _sc_body(cidx_hbm, s_hbm, out_hbm, idx_v, buf0, buf1, sem_g, sem_w):
    wid = lax.axis_index("s") * NUM_CORES + lax.axis_index("c")
    base = wid * ROWS_PER_WORKER

    pltpu.sync_copy(cidx_hbm.at[pl.ds(base, ROWS_PER_WORKER)], idx_v)

    bufs = (buf0, buf1)

    def gather(i):
        return pltpu.async_copy(
            s_hbm.at[idx_v.at[pl.ds(i * CHUNK, CHUNK)]], bufs[i % 2], sem_g)

    def write(i):
        return pltpu.async_copy(
            bufs[i % 2], out_hbm.at[pl.ds(base + i * CHUNK, CHUNK), :], sem_w)

    writes = [None, None]
    h_g = gather(0)
    for i in range(CHUNKS_PER_WORKER):
        if i + 1 < CHUNKS_PER_WORKER:
            if writes[(i + 1) % 2] is not None:
                writes[(i + 1) % 2].wait()
            h_next = gather(i + 1)
        h_g.wait()
        writes[i % 2] = write(i)
        if i + 1 < CHUNKS_PER_WORKER:
            h_g = h_next
    writes[0].wait()
    writes[1].wait()


@jax.jit
def _encode(cidx, s012):
    mesh = plsc.VectorSubcoreMesh(core_axis_name="c", subcore_axis_name="s")
    return pl.kernel(
        _sc_body,
        out_type=jax.ShapeDtypeStruct((NP, HIDDEN), jnp.float32),
        mesh=mesh,
        scratch_types=[
            pltpu.VMEM((ROWS_PER_WORKER,), jnp.int32),
            pltpu.VMEM((CHUNK, HIDDEN), jnp.float32),
            pltpu.VMEM((CHUNK, HIDDEN), jnp.float32),
            pltpu.SemaphoreType.DMA,
            pltpu.SemaphoreType.DMA,
        ],
    )(cidx, s012)


def kernel(x, W0, W1, W2):
    if x.ndim == 1:
        x = x[:, None]
    n = x.shape[0]
    xt = jnp.zeros((3, NP), jnp.int32).at[:, :n].set(x.T.astype(jnp.int32))
    s012, cidx = _combine(W0, W1, W2, xt.reshape(3, NP // HIDDEN, HIDDEN))
    out = _encode(cidx.reshape(NP), s012)
    return out[:n]


# speedup vs baseline: 2.8868x; 1.0169x over previous
import functools

import jax
import jax.numpy as jnp
from jax import lax
from jax.experimental import pallas as pl
from jax.experimental.pallas import tpu as pltpu
from jax.experimental.pallas import tpu_sc as plsc

NUM_CORES = 2
NUM_SUBCORES = 16
NW = NUM_CORES * NUM_SUBCORES

T = 26
HIDDEN = 128
CHUNK = 128
CHUNKS_PER_WORKER = 25
ROWS_PER_WORKER = CHUNK * CHUNKS_PER_WORKER
NP = NW * ROWS_PER_WORKER


def _combine_body(w0_ref, w1_ref, w2_ref, xt_ref, s_ref, s2_ref, cidx_ref):
    w0, w1, w2 = w0_ref[...], w1_ref[...], w2_ref[...]
    s01 = (w0[:, None, :] + w1[None, :, :]).reshape(T * T, HIDDEN)
    s012 = (s01[:, None, :] + w2[None, :, :]).reshape(T * T * T, HIDDEN)
    s_ref[...] = s012
    s2_ref[...] = s012
    xt = xt_ref[...]
    cidx_ref[...] = (T * T) * xt[0] + T * xt[1] + xt[2]


@functools.partial(jax.jit, static_argnames=())
def _combine(w0, w1, w2, xt3):
    return pl.pallas_call(
        _combine_body,
        out_shape=(
            jax.ShapeDtypeStruct((T * T * T, HIDDEN), jnp.float32),
            jax.ShapeDtypeStruct((T * T * T, HIDDEN), jnp.float32),
            jax.ShapeDtypeStruct(xt3.shape[1:], jnp.int32),
        ),
    )(w0, w1, w2, xt3)


def _sc_body(cidx_hbm, s_hbm, s2_hbm, out_hbm, idx_v, buf0, buf1, sem_g, sem_w):
    core = lax.axis_index("c")
    wid = lax.axis_index("s") * NUM_CORES + core
    base = wid * ROWS_PER_WORKER

    pltpu.sync_copy(cidx_hbm.at[pl.ds(base, ROWS_PER_WORKER)], idx_v)

    bufs = (buf0, buf1)

    def pipeline(table_hbm):
        def gather(i):
            return pltpu.async_copy(
                table_hbm.at[idx_v.at[pl.ds(i * CHUNK, CHUNK)]],
                bufs[i % 2], sem_g)

        def write(i):
            return pltpu.async_copy(
                bufs[i % 2], out_hbm.at[pl.ds(base + i * CHUNK, CHUNK), :],
                sem_w)

        writes = [None, None]
        h_g = gather(0)
        for i in range(CHUNKS_PER_WORKER):
            if i + 1 < CHUNKS_PER_WORKER:
                if writes[(i + 1) % 2] is not None:
                    writes[(i + 1) % 2].wait()
                h_next = gather(i + 1)
            h_g.wait()
            writes[i % 2] = write(i)
            if i + 1 < CHUNKS_PER_WORKER:
                h_g = h_next
        writes[0].wait()
        writes[1].wait()

    @pl.when(core == 0)
    def _():
        pipeline(s_hbm)

    @pl.when(core == 1)
    def _():
        pipeline(s2_hbm)


@jax.jit
def _encode(cidx, s012, s012b):
    mesh = plsc.VectorSubcoreMesh(core_axis_name="c", subcore_axis_name="s")
    return pl.kernel(
        _sc_body,
        out_type=jax.ShapeDtypeStruct((NP, HIDDEN), jnp.float32),
        mesh=mesh,
        scratch_types=[
            pltpu.VMEM((ROWS_PER_WORKER,), jnp.int32),
            pltpu.VMEM((CHUNK, HIDDEN), jnp.float32),
            pltpu.VMEM((CHUNK, HIDDEN), jnp.float32),
            pltpu.SemaphoreType.DMA,
            pltpu.SemaphoreType.DMA,
        ],
    )(cidx, s012, s012b)


def kernel(x, W0, W1, W2):
    if x.ndim == 1:
        x = x[:, None]
    n = x.shape[0]
    xt = jnp.zeros((3, NP), jnp.int32).at[:, :n].set(x.T.astype(jnp.int32))
    s012, s012b, cidx = _combine(W0, W1, W2, xt.reshape(3, NP // HIDDEN, HIDDEN))
    out = _encode(cidx.reshape(NP), s012, s012b)
    return out[:n]


# speedup vs baseline: 6.0317x; 2.0894x over previous
import functools

import jax
import jax.numpy as jnp
from jax import lax
from jax.experimental import pallas as pl
from jax.experimental.pallas import tpu as pltpu
from jax.experimental.pallas import tpu_sc as plsc

NUM_CORES = 2
NUM_SUBCORES = 16
NW = NUM_CORES * NUM_SUBCORES
LANES = 16

T = 26
HIDDEN = 128
CHUNK = 128
CHUNKS_PER_WORKER = 25
ROWS_PER_WORKER = CHUNK * CHUNKS_PER_WORKER
NP = NW * ROWS_PER_WORKER
VREGS_PER_ROW = HIDDEN // LANES


def _combine_body(w0_ref, w1_ref, w2_ref, xt_ref, s01_ref, c01_ref, c2_ref):
    w0, w1 = w0_ref[...], w1_ref[...]
    s01_ref[...] = (w0[:, None, :] + w1[None, :, :]).reshape(T * T, HIDDEN)
    xt = xt_ref[...]
    c01_ref[...] = T * xt[0] + xt[1]
    c2_ref[...] = xt[2]
    del w2_ref


@jax.jit
def _combine(w0, w1, w2, xt3):
    return pl.pallas_call(
        _combine_body,
        out_shape=(
            jax.ShapeDtypeStruct((T * T, HIDDEN), jnp.float32),
            jax.ShapeDtypeStruct(xt3.shape[1:], jnp.int32),
            jax.ShapeDtypeStruct(xt3.shape[1:], jnp.int32),
        ),
    )(w0, w1, w2, xt3)


def _sc_body(c01_hbm, c2_hbm, w2_hbm, s01_hbm, out_hbm,
             idx01_v, idx2_v, buf0, buf1, s01_sp, w2_sp,
             sem_g, sem_a, sem_w):
    core = lax.axis_index("c")
    sid = lax.axis_index("s")
    wid = sid * NUM_CORES + core
    base = wid * ROWS_PER_WORKER

    @pl.when(sid == 0)
    def _():
        pltpu.sync_copy(s01_hbm, s01_sp)
        pltpu.sync_copy(w2_hbm, w2_sp)

    pltpu.sync_copy(c01_hbm.at[pl.ds(base, ROWS_PER_WORKER)], idx01_v)
    pltpu.sync_copy(c2_hbm.at[pl.ds(base, ROWS_PER_WORKER)], idx2_v)
    plsc.subcore_barrier()

    bufs = (buf0, buf1)

    def gather01(i):
        return pltpu.async_copy(
            s01_sp.at[idx01_v.at[pl.ds(i * CHUNK, CHUNK)]], bufs[i % 2],
            sem_g)

    def gather2_add(i):
        return pltpu.async_copy(
            w2_sp.at[idx2_v.at[pl.ds(i * CHUNK, CHUNK)]], bufs[i % 2],
            sem_a, add=True)

    def write(i):
        return pltpu.async_copy(
            bufs[i % 2], out_hbm.at[pl.ds(base + i * CHUNK, CHUNK), :], sem_w)

    writes = [None, None]
    h_g = gather01(0)
    for i in range(CHUNKS_PER_WORKER):
        h_g.wait()
        h_a = gather2_add(i)
        if i + 1 < CHUNKS_PER_WORKER:
            if writes[(i + 1) % 2] is not None:
                writes[(i + 1) % 2].wait()
            h_g = gather01(i + 1)
        h_a.wait()
        writes[i % 2] = write(i)
    writes[0].wait()
    writes[1].wait()


@jax.jit
def _encode(c01, c2, w2, s01):
    mesh = plsc.VectorSubcoreMesh(core_axis_name="c", subcore_axis_name="s")
    return pl.kernel(
        _sc_body,
        out_type=jax.ShapeDtypeStruct((NP, HIDDEN), jnp.float32),
        mesh=mesh,
        scratch_types=[
            pltpu.VMEM((ROWS_PER_WORKER,), jnp.int32),
            pltpu.VMEM((ROWS_PER_WORKER,), jnp.int32),
            pltpu.VMEM((CHUNK, HIDDEN), jnp.float32),
            pltpu.VMEM((CHUNK, HIDDEN), jnp.float32),
            pltpu.VMEM_SHARED((T * T, HIDDEN), jnp.float32),
            pltpu.VMEM_SHARED((T, HIDDEN), jnp.float32),
            pltpu.SemaphoreType.DMA,
            pltpu.SemaphoreType.DMA,
            pltpu.SemaphoreType.DMA,
        ],
    )(c01, c2, w2, s01)


def kernel(x, W0, W1, W2):
    if x.ndim == 1:
        x = x[:, None]
    n = x.shape[0]
    xt = jnp.zeros((3, NP), jnp.int32).at[:, :n].set(x.T.astype(jnp.int32))
    s01, c01, c2 = _combine(W0, W1, W2, xt.reshape(3, NP // HIDDEN, HIDDEN))
    out = _encode(c01.reshape(NP), c2.reshape(NP), W2, s01)
    return out[:n]


# speedup vs baseline: 8.9530x; 1.4843x over previous
import jax
import jax.numpy as jnp
from jax import lax
from jax.experimental import pallas as pl
from jax.experimental.pallas import tpu as pltpu
from jax.experimental.pallas import tpu_sc as plsc

NUM_CORES = 2
NUM_SUBCORES = 16
NW = NUM_CORES * NUM_SUBCORES

T = 26
HIDDEN = 128
CHUNK = 128
N_TOTAL = 100000
NUM_CHUNKS = (N_TOTAL + CHUNK - 1) // CHUNK
BIG_WORKERS = NUM_CHUNKS - 24 * NW
MAIN_CHUNKS = 24
SLOTS_PER_WORKER = 25 * CHUNK
NP = 102400
LAST_BASE = N_TOTAL - CHUNK


def _combine_body(w0_ref, w1_ref, w2_ref, xt_ref, s01_ref, c01_ref, c2_ref):
    w0, w1 = w0_ref[...], w1_ref[...]
    s01_ref[...] = (w0[:, None, :] + w1[None, :, :]).reshape(T * T, HIDDEN)
    xt = xt_ref[...]
    c01_ref[...] = T * xt[0] + xt[1]
    c2_ref[...] = xt[2]
    del w2_ref


@jax.jit
def _combine(w0, w1, w2, xt3):
    return pl.pallas_call(
        _combine_body,
        out_shape=(
            jax.ShapeDtypeStruct((T * T, HIDDEN), jnp.float32),
            jax.ShapeDtypeStruct(xt3.shape[1:], jnp.int32),
            jax.ShapeDtypeStruct(xt3.shape[1:], jnp.int32),
        ),
    )(w0, w1, w2, xt3)


def _sc_body(c01_hbm, c2_hbm, w2_hbm, s01_hbm, out_hbm,
             idx01_v, idx2_v, buf0, buf1, s01_sp, w2_sp,
             sem_g, sem_a, sem_w):
    core = lax.axis_index("c")
    sid = lax.axis_index("s")
    wid = sid * NUM_CORES + core
    start = wid * MAIN_CHUNKS + jnp.minimum(wid, BIG_WORKERS)
    sbase = pl.multiple_of(start * CHUNK, CHUNK)

    @pl.when(sid == 0)
    def _():
        pltpu.sync_copy(s01_hbm, s01_sp)
        pltpu.sync_copy(w2_hbm, w2_sp)

    pltpu.sync_copy(c01_hbm.at[pl.ds(sbase, SLOTS_PER_WORKER)], idx01_v)
    pltpu.sync_copy(c2_hbm.at[pl.ds(sbase, SLOTS_PER_WORKER)], idx2_v)
    plsc.subcore_barrier()

    bufs = (buf0, buf1)

    def chunk_base(i):
        ob = jnp.minimum((start + i) * CHUNK, LAST_BASE)
        return pl.multiple_of(ob, 32)

    def gather01(i):
        off = pl.multiple_of(chunk_base(i) - sbase, 32)
        return pltpu.async_copy(
            s01_sp.at[idx01_v.at[pl.ds(off, CHUNK)]], bufs[i % 2], sem_g)

    def gather2_add(i):
        off = pl.multiple_of(chunk_base(i) - sbase, 32)
        return pltpu.async_copy(
            w2_sp.at[idx2_v.at[pl.ds(off, CHUNK)]], bufs[i % 2],
            sem_a, add=True)

    def write(i):
        return pltpu.async_copy(
            bufs[i % 2], out_hbm.at[pl.ds(chunk_base(i), CHUNK), :], sem_w)

    def pipeline(n_chunks):
        writes = [None, None]
        h_g = gather01(0)
        for i in range(n_chunks):
            h_g.wait()
            h_a = gather2_add(i)
            if i + 1 < n_chunks:
                if writes[(i + 1) % 2] is not None:
                    writes[(i + 1) % 2].wait()
                h_g = gather01(i + 1)
            h_a.wait()
            writes[i % 2] = write(i)
        writes[0].wait()
        if n_chunks > 1:
            writes[1].wait()

    @pl.when(wid < BIG_WORKERS)
    def _():
        pipeline(MAIN_CHUNKS + 1)

    @pl.when(wid >= BIG_WORKERS)
    def _():
        pipeline(MAIN_CHUNKS)


@jax.jit
def _encode(c01, c2, w2, s01):
    mesh = plsc.VectorSubcoreMesh(core_axis_name="c", subcore_axis_name="s")
    return pl.kernel(
        _sc_body,
        out_type=jax.ShapeDtypeStruct((N_TOTAL, HIDDEN), jnp.float32),
        mesh=mesh,
        scratch_types=[
            pltpu.VMEM((SLOTS_PER_WORKER,), jnp.int32),
            pltpu.VMEM((SLOTS_PER_WORKER,), jnp.int32),
            pltpu.VMEM((CHUNK, HIDDEN), jnp.float32),
            pltpu.VMEM((CHUNK, HIDDEN), jnp.float32),
            pltpu.VMEM_SHARED((T * T, HIDDEN), jnp.float32),
            pltpu.VMEM_SHARED((T, HIDDEN), jnp.float32),
            pltpu.SemaphoreType.DMA,
            pltpu.SemaphoreType.DMA,
            pltpu.SemaphoreType.DMA,
        ],
    )(c01, c2, w2, s01)


def kernel(x, W0, W1, W2):
    if x.ndim == 1:
        x = x[:, None]
    n = x.shape[0]
    xt = jnp.pad(x.T.astype(jnp.int32), ((0, 0), (0, NP - n)))
    s01, c01, c2 = _combine(W0, W1, W2, xt.reshape(3, NP // HIDDEN, HIDDEN))
    return _encode(c01.reshape(NP), c2.reshape(NP), W2, s01)


# speedup vs baseline: 9.4483x; 1.0553x over previous
import jax
import jax.numpy as jnp
from jax import lax
from jax.experimental import pallas as pl
from jax.experimental.pallas import tpu as pltpu
from jax.experimental.pallas import tpu_sc as plsc

NUM_CORES = 2
NUM_SUBCORES = 16
NW = NUM_CORES * NUM_SUBCORES

T = 26
HIDDEN = 128
CHUNK = 128
N_TOTAL = 100000
NUM_CHUNKS = (N_TOTAL + CHUNK - 1) // CHUNK
BIG_WORKERS = NUM_CHUNKS - 24 * NW
MAIN_CHUNKS = 24
SLOTS_PER_WORKER = 25 * CHUNK
NP = 102400
LAST_BASE = N_TOTAL - CHUNK


def _combine_body(w0_ref, w1_ref, w2_ref, xt_ref, s01_ref, c01_ref, c2_ref):
    w0, w1 = w0_ref[...], w1_ref[...]
    s01_ref[...] = (w0[:, None, :] + w1[None, :, :]).reshape(T * T, HIDDEN)
    xt = xt_ref[...]
    c01_ref[...] = T * xt[0] + xt[1]
    c2_ref[...] = xt[2]
    del w2_ref


@jax.jit
def _combine(w0, w1, w2, xt3):
    return pl.pallas_call(
        _combine_body,
        out_shape=(
            jax.ShapeDtypeStruct((T * T, HIDDEN), jnp.float32),
            jax.ShapeDtypeStruct(xt3.shape[1:], jnp.int32),
            jax.ShapeDtypeStruct(xt3.shape[1:], jnp.int32),
        ),
    )(w0, w1, w2, xt3)


def _sc_body(c01_hbm, c2_hbm, w2_hbm, s01_hbm, out_hbm,
             idx01_v, idx2_v, buf0, buf1, buf2, s01_sp, w2_sp,
             sem_g, sem_a, sem_w0, sem_w1, sem_w2):
    core = lax.axis_index("c")
    sid = lax.axis_index("s")
    wid = sid * NUM_CORES + core
    start = wid * MAIN_CHUNKS + jnp.minimum(wid, BIG_WORKERS)
    sbase = pl.multiple_of(start * CHUNK, CHUNK)

    @pl.when(sid == 0)
    def _():
        pltpu.sync_copy(s01_hbm, s01_sp)
        pltpu.sync_copy(w2_hbm, w2_sp)

    pltpu.sync_copy(c01_hbm.at[pl.ds(sbase, SLOTS_PER_WORKER)], idx01_v)
    pltpu.sync_copy(c2_hbm.at[pl.ds(sbase, SLOTS_PER_WORKER)], idx2_v)
    plsc.subcore_barrier()

    bufs = (buf0, buf1, buf2)
    wsems = (sem_w0, sem_w1, sem_w2)

    def chunk_base(c):
        ob = jnp.minimum((start + c) * CHUNK, LAST_BASE)
        return pl.multiple_of(ob, 32)

    def gather01(c, s):
        off = pl.multiple_of(chunk_base(c) - sbase, 32)
        return pltpu.async_copy(
            s01_sp.at[idx01_v.at[pl.ds(off, CHUNK)]], bufs[s], sem_g)

    def wait_g01(s):
        pltpu.make_async_copy(
            out_hbm.at[pl.ds(0, CHUNK), :], bufs[s], sem_g).wait()

    def gather2_add(c, s):
        off = pl.multiple_of(chunk_base(c) - sbase, 32)
        return pltpu.async_copy(
            w2_sp.at[idx2_v.at[pl.ds(off, CHUNK)]], bufs[s], sem_a, add=True)

    def write(c, s):
        return pltpu.async_copy(
            bufs[s], out_hbm.at[pl.ds(chunk_base(c), CHUNK), :], wsems[s])

    def wait_write(s):
        pltpu.make_async_copy(
            bufs[s], out_hbm.at[pl.ds(0, CHUNK), :], wsems[s]).wait()

    def step(c, s, wait_prev_write):
        wait_g01(s)
        h_a = gather2_add(c, s)
        nxt = (s + 1) % 3
        if wait_prev_write:
            wait_write(nxt)
        gather01(c + 1, nxt)
        h_a.wait()
        write(c, s)

    gather01(0, 0)
    step(0, 0, False)
    step(1, 1, False)
    step(2, 2, True)

    def body(k, carry):
        c = 3 * k
        step(c, 0, True)
        step(c + 1, 1, True)
        step(c + 2, 2, True)
        return carry

    lax.fori_loop(1, 8, body, 0)

    wait_g01(0)

    @pl.when(wid < BIG_WORKERS)
    def _():
        h_a = gather2_add(24, 0)
        h_a.wait()
        write(24, 0)
        wait_write(0)

    wait_write(1)
    wait_write(2)


@jax.jit
def _encode(c01, c2, w2, s01):
    mesh = plsc.VectorSubcoreMesh(core_axis_name="c", subcore_axis_name="s")
    return pl.kernel(
        _sc_body,
        out_type=jax.ShapeDtypeStruct((N_TOTAL, HIDDEN), jnp.float32),
        mesh=mesh,
        scratch_types=[
            pltpu.VMEM((SLOTS_PER_WORKER,), jnp.int32),
            pltpu.VMEM((SLOTS_PER_WORKER,), jnp.int32),
            pltpu.VMEM((CHUNK, HIDDEN), jnp.float32),
            pltpu.VMEM((CHUNK, HIDDEN), jnp.float32),
            pltpu.VMEM((CHUNK, HIDDEN), jnp.float32),
            pltpu.VMEM_SHARED((T * T, HIDDEN), jnp.float32),
            pltpu.VMEM_SHARED((T, HIDDEN), jnp.float32),
            pltpu.SemaphoreType.DMA,
            pltpu.SemaphoreType.DMA,
            pltpu.SemaphoreType.DMA,
            pltpu.SemaphoreType.DMA,
            pltpu.SemaphoreType.DMA,
        ],
    )(c01, c2, w2, s01)


def kernel(x, W0, W1, W2):
    if x.ndim == 1:
        x = x[:, None]
    n = x.shape[0]
    xt = jnp.pad(x.T.astype(jnp.int32), ((0, 0), (0, NP - n)))
    s01, c01, c2 = _combine(W0, W1, W2, xt.reshape(3, NP // HIDDEN, HIDDEN))
    return _encode(c01.reshape(NP), c2.reshape(NP), W2, s01)
